# baseline (speedup 1.0000x reference)
import jax
import jax.numpy as jnp
from jax import lax
from jax.experimental import pallas as pl

NUM_CLASSES = 1000
BR = 1024


def _onehot_body(x_ref, o_ref):
    idx = x_ref[...]
    iota = lax.broadcasted_iota(jnp.int32, (BR, NUM_CLASSES), 1)
    o_ref[...] = (idx == iota).astype(jnp.int32)


def kernel(x1):
    B, C = x1.shape
    n = B * C
    idx_col = x1.reshape(n, 1)
    out = pl.pallas_call(
        _onehot_body,
        grid=(n // BR,),
        in_specs=[pl.BlockSpec((BR, 1), lambda i: (i, 0))],
        out_specs=pl.BlockSpec((BR, NUM_CLASSES), lambda i: (i, 0)),
        out_shape=jax.ShapeDtypeStruct((n, NUM_CLASSES), jnp.int32),
    )(idx_col)
    return out.reshape(B, C, NUM_CLASSES)


# speedup vs baseline: 1.4630x; 1.4630x over previous
import jax
import jax.numpy as jnp
from jax import lax
from jax.experimental import pallas as pl

NUM_CLASSES = 1000
BR = 128


def _onehot_body(x_ref, o_ref):
    idx = x_ref[...]
    iota = lax.broadcasted_iota(jnp.int32, (BR, x_ref.shape[1], NUM_CLASSES), 2)
    o_ref[...] = (idx[:, :, None] == iota).astype(jnp.int32)


def kernel(x1):
    B, C = x1.shape
    out = pl.pallas_call(
        _onehot_body,
        grid=(B // BR,),
        in_specs=[pl.BlockSpec((BR, C), lambda i: (i, 0))],
        out_specs=pl.BlockSpec((BR, C, NUM_CLASSES), lambda i: (i, 0, 0)),
        out_shape=jax.ShapeDtypeStruct((B, C, NUM_CLASSES), jnp.int32),
    )(x1)
    return out


# speedup vs baseline: 1.4680x; 1.0034x over previous
import jax
import jax.numpy as jnp
from jax import lax
from jax.experimental import pallas as pl
from jax.experimental.pallas import tpu as pltpu

NUM_CLASSES = 1000
BR = 32
NBUF = 4


def _onehot_body(x_ref, o_hbm, buf, sems):
    i = pl.program_id(0)
    nsteps = pl.num_programs(0)
    slot = lax.rem(i, NBUF)
    C = x_ref.shape[1]

    @pl.when(i >= NBUF)
    def _wait_prev():
        pltpu.make_async_copy(
            buf.at[slot], o_hbm.at[pl.ds((i - NBUF) * BR, BR)], sems.at[slot]
        ).wait()

    idx = x_ref[...]
    iota = lax.broadcasted_iota(jnp.int32, (BR, C, NUM_CLASSES), 2)
    buf[slot] = (idx[:, :, None] == iota).astype(jnp.int32)

    pltpu.make_async_copy(
        buf.at[slot], o_hbm.at[pl.ds(i * BR, BR)], sems.at[slot]
    ).start()

    @pl.when(i == nsteps - 1)
    def _drain():
        for j in range(NBUF):
            pltpu.make_async_copy(
                buf.at[j], o_hbm.at[pl.ds(0, BR)], sems.at[j]
            ).wait()


def kernel(x1):
    B, C = x1.shape
    out = pl.pallas_call(
        _onehot_body,
        grid=(B // BR,),
        in_specs=[pl.BlockSpec((BR, C), lambda i: (i, 0))],
        out_specs=pl.BlockSpec(memory_space=pl.ANY),
        out_shape=jax.ShapeDtypeStruct((B, C, NUM_CLASSES), jnp.int32),
        scratch_shapes=[
            pltpu.VMEM((NBUF, BR, C, NUM_CLASSES), jnp.int32),
            pltpu.SemaphoreType.DMA((NBUF,)),
        ],
    )(x1)
    return out
